# baseline (speedup 1.0000x reference)
import functools

import jax
import jax.numpy as jnp
from jax import lax
from jax.experimental import pallas as pl
from jax.experimental.pallas import tpu as pltpu
from jax.experimental.pallas import tpu_sc as plsc

BOHR_AU = 1.8897261258369282
N_PAIRS = 6400000
SPECIES = 90
N_POLY = 12

NUM_CORES = 2
NUM_SUBCORES = 16
LANES = 16
NW = NUM_CORES * NUM_SUBCORES

SC_N = 1600000
PER_W = SC_N // NW
SC_CHUNK = 10000
SC_NCHUNK = PER_W // SC_CHUNK
UNROLL = 5
GROUPS = SC_CHUNK // (UNROLL * LANES)

TC_N = N_PAIRS - SC_N
COLS = 128
ROWS = TC_N // COLS
CH_R = 625
NCH = ROWS // CH_R
NBUF = 8


def _sc_poly(dr_head, spec16, param_pad):
    mesh = plsc.VectorSubcoreMesh(core_axis_name="c", subcore_axis_name="s")

    @functools.partial(
        pl.kernel,
        out_type=jax.ShapeDtypeStruct((SC_N,), jnp.float32),
        mesh=mesh,
        compiler_params=pltpu.CompilerParams(needs_layout_passes=False),
        scratch_types=[
            pltpu.VMEM((LANES,), jnp.int32),
            pltpu.VMEM((LANES,), jnp.float32),
            pltpu.VMEM((SC_CHUNK,), jnp.float32),
            pltpu.VMEM((SC_CHUNK,), jnp.float32),
            pltpu.VMEM((SC_CHUNK,), jnp.float32),
            pltpu.VMEM((SC_CHUNK,), jnp.float32),
            pltpu.SemaphoreType.DMA,
            pltpu.SemaphoreType.DMA,
            pltpu.SemaphoreType.DMA,
            pltpu.SemaphoreType.DMA,
            pltpu.SemaphoreType.DMA,
        ],
    )
    def poly_kernel(dr_hbm, spec_hbm, param_hbm, out_hbm,
                    spec_v, coef_v, xb0, xb1, yb0, yb1,
                    sem_p, sem_i0, sem_i1, sem_o0, sem_o1):
        pltpu.sync_copy(spec_hbm, spec_v)
        sv = spec_v[...]
        flat = sv[0] * SPECIES + sv[1]
        pltpu.async_copy(
            param_hbm.at[pl.ds(flat * LANES, LANES)], coef_v, sem_p).wait()
        cs = [
            plsc.load_gather(coef_v, [jnp.full((LANES,), k, jnp.int32)])
            * jnp.float32(BOHR_AU ** k)
            for k in range(N_POLY)
        ]

        wid = lax.axis_index("c") * NUM_SUBCORES + lax.axis_index("s")
        base = wid * PER_W

        xbufs = [xb0, xb1]
        ybufs = [yb0, yb1]
        sem_in = [sem_i0, sem_i1]
        sem_out = [sem_o0, sem_o1]

        def compute_chunk(xref, yref):
            def body(g, carry):
                b = g * (UNROLL * LANES)
                xs = [xref[pl.ds(b + u * LANES, LANES)] for u in range(UNROLL)]
                ys = [cs[N_POLY - 1]] * UNROLL
                for k in range(N_POLY - 2, -1, -1):
                    ck = cs[k]
                    ys = [y * x + ck for y, x in zip(ys, xs)]
                for u in range(UNROLL):
                    yref[pl.ds(b + u * LANES, LANES)] = ys[u]
                return carry
            lax.fori_loop(0, GROUPS, body, 0)

        in_d = [None, None]
        out_d = [None, None]
        in_d[0] = pltpu.async_copy(
            dr_hbm.at[pl.ds(base, SC_CHUNK)], xbufs[0], sem_in[0])
        for c in range(SC_NCHUNK):
            b = c % 2
            nb = (c + 1) % 2
            if c + 1 < SC_NCHUNK:
                in_d[nb] = pltpu.async_copy(
                    dr_hbm.at[pl.ds(base + (c + 1) * SC_CHUNK, SC_CHUNK)],
                    xbufs[nb], sem_in[nb])
            in_d[b].wait()
            if out_d[b] is not None:
                out_d[b].wait()
            compute_chunk(xbufs[b], ybufs[b])
            out_d[b] = pltpu.async_copy(
                ybufs[b], out_hbm.at[pl.ds(base + c * SC_CHUNK, SC_CHUNK)],
                sem_out[b])
        out_d[(SC_NCHUNK - 2) % 2].wait()
        out_d[(SC_NCHUNK - 1) % 2].wait()

    return poly_kernel(dr_head, spec16, param_pad)


def _sc_gather_row(spec16, param_pad):
    mesh = plsc.VectorSubcoreMesh(core_axis_name="c", subcore_axis_name="s")

    @functools.partial(
        pl.kernel,
        out_type=jax.ShapeDtypeStruct((LANES,), jnp.float32),
        mesh=mesh,
        compiler_params=pltpu.CompilerParams(needs_layout_passes=False),
        scratch_types=[
            pltpu.VMEM((LANES,), jnp.int32),
            pltpu.VMEM((LANES,), jnp.float32),
            pltpu.SemaphoreType.DMA,
        ],
    )
    def gather_kernel(spec_hbm, param_hbm, out_hbm, spec_v, row_v, sem):
        wid = lax.axis_index("c") * NUM_SUBCORES + lax.axis_index("s")

        @pl.when(wid == 0)
        def _():
            pltpu.sync_copy(spec_hbm, spec_v)
            sv = spec_v[...]
            flat = sv[0] * SPECIES + sv[1]
            pltpu.async_copy(
                param_hbm.at[pl.ds(flat * LANES, LANES)], row_v, sem).wait()
            pltpu.sync_copy(row_v, out_hbm)

    return gather_kernel(spec16, param_pad)


def _tc_horner(x2d, row):
    def body(row_ref, x_hbm, o_hbm, xb, yb, sem_in, sem_out):
        cs = [row_ref[k] * jnp.float32(BOHR_AU ** k) for k in range(N_POLY)]

        def in_dma(b, c):
            return pltpu.make_async_copy(
                x_hbm.at[pl.ds(c * CH_R, CH_R)], xb.at[b], sem_in.at[b])

        def out_dma(b, c):
            return pltpu.make_async_copy(
                yb.at[b], o_hbm.at[pl.ds(c * CH_R, CH_R)], sem_out.at[b])

        for b in range(NBUF):
            in_dma(b, b).start()
        for c in range(NCH):
            b = c % NBUF
            in_dma(b, c).wait()
            if c >= NBUF:
                out_dma(b, c - NBUF).wait()
            x = xb[b]
            y = jnp.full(x.shape, cs[N_POLY - 1])
            for k in range(N_POLY - 2, -1, -1):
                y = y * x + cs[k]
            yb[b] = y
            out_dma(b, c).start()
            if c + NBUF < NCH:
                in_dma(b, c + NBUF).start()
        for c in range(NCH - NBUF, NCH):
            out_dma(c % NBUF, c).wait()

    return pl.pallas_call(
        body,
        in_specs=[
            pl.BlockSpec(memory_space=pltpu.SMEM),
            pl.BlockSpec(memory_space=pltpu.HBM),
        ],
        out_specs=pl.BlockSpec(memory_space=pltpu.HBM),
        out_shape=jax.ShapeDtypeStruct((ROWS, COLS), jnp.float32),
        scratch_shapes=[
            pltpu.VMEM((NBUF, CH_R, COLS), jnp.float32),
            pltpu.VMEM((NBUF, CH_R, COLS), jnp.float32),
            pltpu.SemaphoreType.DMA((NBUF,)),
            pltpu.SemaphoreType.DMA((NBUF,)),
        ],
    )(row, x2d)


def kernel(dr, species_tuple, param_matrix):
    spec16 = jnp.zeros((LANES,), jnp.int32).at[:2].set(
        species_tuple.astype(jnp.int32))
    param_pad = jnp.pad(
        param_matrix.reshape(SPECIES * SPECIES, N_POLY),
        ((0, 0), (0, LANES - N_POLY))).reshape(-1)
    row = _sc_gather_row(spec16, param_pad)
    y_head = _sc_poly(dr[:SC_N], spec16, param_pad)
    y_tail = _tc_horner(dr[SC_N:].reshape(ROWS, COLS), row)
    return jnp.concatenate([y_head, y_tail.reshape(-1)])


# speedup vs baseline: 1.7125x; 1.7125x over previous
import functools

import jax
import jax.numpy as jnp
from jax import lax
from jax.experimental import pallas as pl
from jax.experimental.pallas import tpu as pltpu
from jax.experimental.pallas import tpu_sc as plsc

BOHR_AU = 1.8897261258369282
N_PAIRS = 6400000
SPECIES = 90
N_POLY = 12

NUM_CORES = 2
NUM_SUBCORES = 16
LANES = 16

ROWS = 50000
COLS = 128
CH_R = 1250
NCH = ROWS // CH_R
NBUF = 4


def _sc_gather_row(spec16, param_pad):
    mesh = plsc.VectorSubcoreMesh(core_axis_name="c", subcore_axis_name="s")

    @functools.partial(
        pl.kernel,
        out_type=jax.ShapeDtypeStruct((LANES,), jnp.float32),
        mesh=mesh,
        compiler_params=pltpu.CompilerParams(needs_layout_passes=False),
        scratch_types=[
            pltpu.VMEM((LANES,), jnp.int32),
            pltpu.VMEM((LANES,), jnp.float32),
            pltpu.SemaphoreType.DMA,
        ],
    )
    def gather_kernel(spec_hbm, param_hbm, out_hbm, spec_v, row_v, sem):
        wid = lax.axis_index("c") * NUM_SUBCORES + lax.axis_index("s")

        @pl.when(wid == 0)
        def _():
            pltpu.sync_copy(spec_hbm, spec_v)
            sv = spec_v[...]
            flat = sv[0] * SPECIES + sv[1]
            pltpu.async_copy(
                param_hbm.at[pl.ds(flat * LANES, LANES)], row_v, sem).wait()
            pltpu.sync_copy(row_v, out_hbm)

    return gather_kernel(spec16, param_pad)


def _tc_horner(x2d, row):
    def body(row_ref, x_hbm, o_hbm, xb, yb, sem_in, sem_out):
        cs = [row_ref[k] * jnp.float32(BOHR_AU ** k) for k in range(N_POLY)]

        def in_dma(b, c):
            return pltpu.make_async_copy(
                x_hbm.at[pl.ds(c * CH_R, CH_R)], xb.at[b], sem_in.at[b])

        def out_dma(b, c):
            return pltpu.make_async_copy(
                yb.at[b], o_hbm.at[pl.ds(c * CH_R, CH_R)], sem_out.at[b])

        for b in range(NBUF):
            in_dma(b, b).start()
        for c in range(NCH):
            b = c % NBUF
            in_dma(b, c).wait()
            if c >= NBUF:
                out_dma(b, c - NBUF).wait()
            x = xb[b]
            y = jnp.full(x.shape, cs[N_POLY - 1])
            for k in range(N_POLY - 2, -1, -1):
                y = y * x + cs[k]
            yb[b] = y
            out_dma(b, c).start()
            if c + NBUF < NCH:
                in_dma(b, c + NBUF).start()
        for c in range(NCH - NBUF, NCH):
            out_dma(c % NBUF, c).wait()

    return pl.pallas_call(
        body,
        in_specs=[
            pl.BlockSpec(memory_space=pltpu.SMEM),
            pl.BlockSpec(memory_space=pltpu.HBM),
        ],
        out_specs=pl.BlockSpec(memory_space=pltpu.HBM),
        out_shape=jax.ShapeDtypeStruct((ROWS, COLS), jnp.float32),
        scratch_shapes=[
            pltpu.VMEM((NBUF, CH_R, COLS), jnp.float32),
            pltpu.VMEM((NBUF, CH_R, COLS), jnp.float32),
            pltpu.SemaphoreType.DMA((NBUF,)),
            pltpu.SemaphoreType.DMA((NBUF,)),
        ],
    )(row, x2d)


def kernel(dr, species_tuple, param_matrix):
    spec16 = jnp.zeros((LANES,), jnp.int32).at[:2].set(
        species_tuple.astype(jnp.int32))
    param_pad = jnp.pad(
        param_matrix.reshape(SPECIES * SPECIES, N_POLY),
        ((0, 0), (0, LANES - N_POLY))).reshape(-1)
    row = _sc_gather_row(spec16, param_pad)
    y2d = _tc_horner(dr.reshape(ROWS, COLS), row)
    return y2d.reshape(-1)


# speedup vs baseline: 1.7554x; 1.0250x over previous
import functools

import jax
import jax.numpy as jnp
from jax import lax
from jax.experimental import pallas as pl
from jax.experimental.pallas import tpu as pltpu
from jax.experimental.pallas import tpu_sc as plsc

BOHR_AU = 1.8897261258369282
N_PAIRS = 6400000
SPECIES = 90
N_POLY = 12

NUM_CORES = 2
NUM_SUBCORES = 16
LANES = 16

ROWS = 50000
COLS = 128
CH_R = 500
NCH = ROWS // CH_R
NBUF = 10


def _sc_gather_row(spec16, param_pad):
    mesh = plsc.VectorSubcoreMesh(core_axis_name="c", subcore_axis_name="s")

    @functools.partial(
        pl.kernel,
        out_type=jax.ShapeDtypeStruct((LANES,), jnp.float32),
        mesh=mesh,
        compiler_params=pltpu.CompilerParams(needs_layout_passes=False),
        scratch_types=[
            pltpu.VMEM((LANES,), jnp.int32),
            pltpu.VMEM((LANES,), jnp.float32),
            pltpu.SemaphoreType.DMA,
        ],
    )
    def gather_kernel(spec_hbm, param_hbm, out_hbm, spec_v, row_v, sem):
        wid = lax.axis_index("c") * NUM_SUBCORES + lax.axis_index("s")

        @pl.when(wid == 0)
        def _():
            pltpu.sync_copy(spec_hbm, spec_v)
            sv = spec_v[...]
            flat = sv[0] * SPECIES + sv[1]
            pltpu.async_copy(
                param_hbm.at[pl.ds(flat * LANES, LANES)], row_v, sem).wait()
            pltpu.sync_copy(row_v, out_hbm)

    return gather_kernel(spec16, param_pad)


def _tc_horner(x2d, row):
    def body(row_ref, x_hbm, o_hbm, xb, yb, sem_in, sem_out):
        cs = [row_ref[k] * jnp.float32(BOHR_AU ** k) for k in range(N_POLY)]

        def in_dma(b, c):
            return pltpu.make_async_copy(
                x_hbm.at[pl.ds(c * CH_R, CH_R)], xb.at[b], sem_in.at[b])

        def out_dma(b, c):
            return pltpu.make_async_copy(
                yb.at[b], o_hbm.at[pl.ds(c * CH_R, CH_R)], sem_out.at[b])

        for b in range(NBUF):
            in_dma(b, b).start()
        for c in range(NCH):
            b = c % NBUF
            in_dma(b, c).wait()
            if c >= NBUF:
                out_dma(b, c - NBUF).wait()
            x = xb[b]
            y = jnp.full(x.shape, cs[N_POLY - 1])
            for k in range(N_POLY - 2, -1, -1):
                y = y * x + cs[k]
            yb[b] = y
            out_dma(b, c).start()
            if c + NBUF < NCH:
                in_dma(b, c + NBUF).start()
        for c in range(NCH - NBUF, NCH):
            out_dma(c % NBUF, c).wait()

    return pl.pallas_call(
        body,
        in_specs=[
            pl.BlockSpec(memory_space=pltpu.SMEM),
            pl.BlockSpec(memory_space=pltpu.HBM),
        ],
        out_specs=pl.BlockSpec(memory_space=pltpu.HBM),
        out_shape=jax.ShapeDtypeStruct((ROWS, COLS), jnp.float32),
        scratch_shapes=[
            pltpu.VMEM((NBUF, CH_R, COLS), jnp.float32),
            pltpu.VMEM((NBUF, CH_R, COLS), jnp.float32),
            pltpu.SemaphoreType.DMA((NBUF,)),
            pltpu.SemaphoreType.DMA((NBUF,)),
        ],
    )(row, x2d)


def kernel(dr, species_tuple, param_matrix):
    spec16 = jnp.zeros((LANES,), jnp.int32).at[:2].set(
        species_tuple.astype(jnp.int32))
    param_pad = jnp.pad(
        param_matrix.reshape(SPECIES * SPECIES, N_POLY),
        ((0, 0), (0, LANES - N_POLY))).reshape(-1)
    row = _sc_gather_row(spec16, param_pad)
    y2d = _tc_horner(dr.reshape(ROWS, COLS), row)
    return y2d.reshape(-1)
